# baseline (speedup 1.0000x reference)
import functools

import jax
import jax.numpy as jnp
from jax import lax
from jax.experimental import pallas as pl
from jax.experimental.pallas import tpu as pltpu
from jax.experimental.pallas import tpu_sc as plsc

VOCAB = 1000000
MAXLEN = 200
EMBED = 32
BATCH = 4096

_ROWS = BATCH * MAXLEN
_G = 128
_TB = 16384
_NTB = pl.cdiv(VOCAB, _TB)
_TROWS = _NTB * (_TB // 4)

_info = plsc.get_sparse_core_info()
_NC, _NS = _info.num_cores, _info.num_subcores
_NW = _NC * _NS
_B_PER_W = _ROWS // _NW
_CHUNK = 1024
_NCHUNK = _B_PER_W // _CHUNK
_NG = _CHUNK // _G


def _t1_body(tin_ref, tout_ref):
    eye = jnp.eye(EMBED, dtype=jnp.float32)
    qn = _TB // 4
    parts = []
    for q in range(4):
        a = tin_ref[:, pl.ds(q * qn, qn)]
        parts.append(
            jax.lax.dot_general(a, eye, (((0,), (0,)), ((), ())),
                                precision=jax.lax.Precision.HIGHEST,
                                preferred_element_type=jnp.float32))
    tout_ref[...] = jnp.concatenate(parts, axis=1)


_t1 = pl.pallas_call(
    _t1_body,
    grid=(_NTB,),
    in_specs=[pl.BlockSpec((EMBED, _TB), lambda j: (0, j))],
    out_specs=pl.BlockSpec((_TB // 4, 128), lambda j: (j, 0)),
    out_shape=jax.ShapeDtypeStruct((_TROWS, 128), jnp.float32),
)


@functools.partial(
    pl.kernel,
    out_type=jax.ShapeDtypeStruct((_ROWS * EMBED // 128, 128), jnp.float32),
    mesh=plsc.VectorSubcoreMesh(core_axis_name="c", subcore_axis_name="s"),
    compiler_params=pltpu.CompilerParams(use_tc_tiling_on_sc=False),
    scratch_types=[
        pltpu.VMEM((_CHUNK,), jnp.int32),
        pltpu.VMEM((_CHUNK,), jnp.int32),
        pltpu.VMEM((_CHUNK, EMBED), jnp.float32),
        pltpu.VMEM((MAXLEN * EMBED,), jnp.float32),
        pltpu.SemaphoreType.DMA,
    ],
)
def _k2(x_hbm, tbl_hbm, pos_hbm, out_hbm, idx_v, row_v, rows_v, pos_v, sem):
    wid = lax.axis_index("s") * _NC + lax.axis_index("c")
    base = wid * _B_PER_W

    pltpu.sync_copy(pos_hbm, pos_v)

    for c in range(_NCHUNK):
        cb = base + c * _CHUNK
        l = cb >> 12
        q = lax.shift_right_logical(cb, 10) & 3
        pltpu.sync_copy(x_hbm.at[pl.ds(cb, _CHUNK)], idx_v)

        def _to_rows(v, carry):
            t = idx_v[pl.ds(v * 16, 16)]
            r = (
                jnp.bitwise_and(t, jnp.int32(~(_TB - 1)))
                | lax.shift_left(jnp.bitwise_and(t, _TB // 4 - 1), 2)
                | jnp.bitwise_and(
                    lax.shift_right_logical(t, (_TB // 4).bit_length() - 1), 3)
            )
            row_v[pl.ds(v * 16, 16)] = r
            return carry

        lax.fori_loop(0, _CHUNK // 16, _to_rows, 0)

        copies = []
        for j in range(_NG):
            copies.append(
                pltpu.async_copy(
                    tbl_hbm.at[row_v.at[pl.ds(j * _G, _G)]],
                    rows_v.at[pl.ds(j * _G, _G)],
                    sem,
                )
            )
        for cp in copies:
            cp.wait()

        pv0 = pos_v[pl.ds(l * EMBED, 16)]
        pv1 = pos_v[pl.ds(l * EMBED + 16, 16)]

        def _add_pos(g, carry):
            for s in range(16):
                r = g * 16 + s
                plsc.addupdate(rows_v.at[r, pl.ds(0, 16)], pv0)
                plsc.addupdate(rows_v.at[r, pl.ds(16, 16)], pv1)
            return carry

        lax.fori_loop(0, _CHUNK // 16, _add_pos, 0)

        pltpu.sync_copy(
            rows_v,
            out_hbm.at[pl.ds(l * 1024, 1024), pl.ds(q * EMBED, EMBED)])


_LPB = 2


def _t3_body(gin_ref, out_ref):
    eye = jnp.eye(EMBED, dtype=jnp.float32)
    for i in range(_LPB):
        g = gin_ref[pl.ds(i * 1024, 1024), :]
        toks = jnp.concatenate(
            [g[:, q * EMBED:(q + 1) * EMBED] for q in range(4)], axis=0)
        t = jax.lax.dot_general(eye, toks, (((1,), (1,)), ((), ())),
                                precision=jax.lax.Precision.HIGHEST,
                                preferred_element_type=jnp.float32)
        r = t.reshape(EMBED // 8, 8, BATCH // 128, 128)
        out_ref[i] = r.transpose(0, 2, 1, 3)


_t3 = pl.pallas_call(
    _t3_body,
    grid=(MAXLEN // _LPB,),
    in_specs=[
        pl.BlockSpec((_LPB * BATCH * EMBED // 128, 128), lambda l: (l, 0)),
    ],
    out_specs=pl.BlockSpec(
        (_LPB, EMBED // 8, BATCH // 128, 8, 128), lambda l: (l, 0, 0, 0, 0)),
    out_shape=jax.ShapeDtypeStruct(
        (MAXLEN, EMBED // 8, BATCH // 128, 8, 128), jnp.float32),
)


def kernel(x, token_table, pos_table):
    tbl = _t1(token_table.T).reshape(_TROWS * 4, EMBED)
    xf = x.T.astype(jnp.int32).reshape(_ROWS)
    posf = pos_table.reshape(MAXLEN * EMBED)
    g = _k2(xf, tbl, posf)
    out5 = _t3(g)
    return out5.transpose(2, 4, 0, 1, 3).reshape(BATCH, MAXLEN, EMBED)


# speedup vs baseline: 2.0693x; 2.0693x over previous
import functools

import jax
import jax.numpy as jnp
from jax import lax
from jax.experimental import pallas as pl
from jax.experimental.pallas import tpu as pltpu
from jax.experimental.pallas import tpu_sc as plsc

VOCAB = 1000000
MAXLEN = 200
EMBED = 32
BATCH = 4096

_ROWS = BATCH * MAXLEN
_G = 128
_TB = 16384
_NTB = pl.cdiv(VOCAB, _TB)
_TROWS = _NTB * (_TB // 4)

_info = plsc.get_sparse_core_info()
_NC, _NS = _info.num_cores, _info.num_subcores
_NW = _NC * _NS
_B_PER_W = _ROWS // _NW
_CHUNK = 1024
_NCHUNK = _B_PER_W // _CHUNK
_NG = _CHUNK // _G


def _t1_body(tin_ref, tout_ref):
    eye = jnp.eye(EMBED, dtype=jnp.float32)
    qn = _TB // 4
    parts = []
    for q in range(4):
        a = tin_ref[:, pl.ds(q * qn, qn)]
        parts.append(
            jax.lax.dot_general(a, eye, (((0,), (0,)), ((), ())),
                                preferred_element_type=jnp.float32))
    tout_ref[...] = jnp.concatenate(parts, axis=1)


_t1 = pl.pallas_call(
    _t1_body,
    grid=(_NTB,),
    in_specs=[pl.BlockSpec((EMBED, _TB), lambda j: (0, j))],
    out_specs=pl.BlockSpec((_TB // 4, 128), lambda j: (j, 0)),
    out_shape=jax.ShapeDtypeStruct((_TROWS, 128), jnp.float32),
)


@functools.partial(
    pl.kernel,
    out_type=jax.ShapeDtypeStruct((_ROWS * EMBED // 128, 128), jnp.float32),
    mesh=plsc.VectorSubcoreMesh(core_axis_name="c", subcore_axis_name="s"),
    compiler_params=pltpu.CompilerParams(use_tc_tiling_on_sc=False),
    scratch_types=[
        pltpu.VMEM((_CHUNK,), jnp.int32),
        pltpu.VMEM((_CHUNK,), jnp.int32),
        pltpu.VMEM((_CHUNK, EMBED), jnp.float32),
        pltpu.VMEM((MAXLEN * EMBED,), jnp.float32),
        pltpu.SemaphoreType.DMA,
    ],
)
def _k2(x_hbm, tbl_hbm, pos_hbm, out_hbm, idx_v, row_v, rows_v, pos_v, sem):
    wid = lax.axis_index("s") * _NC + lax.axis_index("c")
    base = wid * _B_PER_W

    pltpu.sync_copy(pos_hbm, pos_v)

    for c in range(_NCHUNK):
        cb = base + c * _CHUNK
        l = cb >> 12
        q = lax.shift_right_logical(cb, 10) & 3
        pltpu.sync_copy(x_hbm.at[pl.ds(cb, _CHUNK)], idx_v)

        def _to_rows(v, carry):
            t = idx_v[pl.ds(v * 16, 16)]
            r = (
                jnp.bitwise_and(t, jnp.int32(~(_TB - 1)))
                | lax.shift_left(jnp.bitwise_and(t, _TB // 4 - 1), 2)
                | jnp.bitwise_and(
                    lax.shift_right_logical(t, (_TB // 4).bit_length() - 1), 3)
            )
            row_v[pl.ds(v * 16, 16)] = r
            return carry

        lax.fori_loop(0, _CHUNK // 16, _to_rows, 0)

        copies = []
        for j in range(_NG):
            copies.append(
                pltpu.async_copy(
                    tbl_hbm.at[row_v.at[pl.ds(j * _G, _G)]],
                    rows_v.at[pl.ds(j * _G, _G)],
                    sem,
                )
            )
        for cp in copies:
            cp.wait()

        pv0 = pos_v[pl.ds(l * EMBED, 16)]
        pv1 = pos_v[pl.ds(l * EMBED + 16, 16)]

        def _add_pos(g, carry):
            for s in range(16):
                r = g * 16 + s
                plsc.addupdate(rows_v.at[r, pl.ds(0, 16)], pv0)
                plsc.addupdate(rows_v.at[r, pl.ds(16, 16)], pv1)
            return carry

        lax.fori_loop(0, _CHUNK // 16, _add_pos, 0)

        pltpu.sync_copy(
            rows_v,
            out_hbm.at[pl.ds(l * 1024, 1024), pl.ds(q * EMBED, EMBED)])


_LPB = 2


def _t3_body(gin_ref, out_ref):
    eye = jnp.eye(EMBED, dtype=jnp.float32)
    for i in range(_LPB):
        g = gin_ref[pl.ds(i * 1024, 1024), :]
        toks = jnp.concatenate(
            [g[:, q * EMBED:(q + 1) * EMBED] for q in range(4)], axis=0)
        t = jax.lax.dot_general(eye, toks, (((1,), (1,)), ((), ())),
                                preferred_element_type=jnp.float32)
        r = t.reshape(EMBED // 8, 8, BATCH // 128, 128)
        out_ref[i] = r.transpose(0, 2, 1, 3)


_t3 = pl.pallas_call(
    _t3_body,
    grid=(MAXLEN // _LPB,),
    in_specs=[
        pl.BlockSpec((_LPB * BATCH * EMBED // 128, 128), lambda l: (l, 0)),
    ],
    out_specs=pl.BlockSpec(
        (_LPB, EMBED // 8, BATCH // 128, 8, 128), lambda l: (l, 0, 0, 0, 0)),
    out_shape=jax.ShapeDtypeStruct(
        (MAXLEN, EMBED // 8, BATCH // 128, 8, 128), jnp.float32),
)


def kernel(x, token_table, pos_table):
    tbl = _t1(token_table.T).reshape(_TROWS * 4, EMBED)
    xf = x.T.astype(jnp.int32).reshape(_ROWS)
    posf = pos_table.reshape(MAXLEN * EMBED)
    g = _k2(xf, tbl, posf)
    out5 = _t3(g)
    return out5.transpose(2, 4, 0, 1, 3).reshape(BATCH, MAXLEN, EMBED)


# speedup vs baseline: 2.4737x; 1.1954x over previous
import functools

import jax
import jax.numpy as jnp
from jax import lax
from jax.experimental import pallas as pl
from jax.experimental.pallas import tpu as pltpu
from jax.experimental.pallas import tpu_sc as plsc

VOCAB = 1000000
MAXLEN = 200
EMBED = 32
BATCH = 4096

_ROWS = BATCH * MAXLEN
_G = 128
_TB = 16384
_NTB = pl.cdiv(VOCAB, _TB)
_TROWS = _NTB * (_TB // 4)

_info = plsc.get_sparse_core_info()
_NC, _NS = _info.num_cores, _info.num_subcores
_NW = _NC * _NS
_B_PER_W = _ROWS // _NW
_CHUNK = 1024
_NCHUNK = _B_PER_W // _CHUNK
_NG = _CHUNK // _G


def _t1_body(tin_ref, tout_ref):
    eye = jnp.eye(EMBED, dtype=jnp.float32)
    qn = _TB // 4
    parts = []
    for q in range(4):
        a = tin_ref[:, pl.ds(q * qn, qn)]
        parts.append(
            jax.lax.dot_general(a, eye, (((0,), (0,)), ((), ())),
                                preferred_element_type=jnp.float32))
    tout_ref[...] = jnp.concatenate(parts, axis=1)


_t1 = pl.pallas_call(
    _t1_body,
    grid=(_NTB,),
    in_specs=[pl.BlockSpec((EMBED, _TB), lambda j: (0, j))],
    out_specs=pl.BlockSpec((_TB // 4, 128), lambda j: (j, 0)),
    out_shape=jax.ShapeDtypeStruct((_TROWS, 128), jnp.float32),
)


@functools.partial(
    pl.kernel,
    out_type=jax.ShapeDtypeStruct((_ROWS * EMBED // 128, 128), jnp.float32),
    mesh=plsc.VectorSubcoreMesh(core_axis_name="c", subcore_axis_name="s"),
    compiler_params=pltpu.CompilerParams(use_tc_tiling_on_sc=False),
    scratch_types=[
        pltpu.VMEM((2, _CHUNK), jnp.int32),
        pltpu.VMEM((2, _CHUNK), jnp.int32),
        pltpu.VMEM((2, _CHUNK, EMBED), jnp.float32),
        pltpu.VMEM((MAXLEN * EMBED,), jnp.float32),
        pltpu.SemaphoreType.DMA,
        pltpu.SemaphoreType.DMA,
        pltpu.SemaphoreType.DMA,
        pltpu.SemaphoreType.DMA,
    ],
)
def _k2(x_hbm, tbl_hbm, pos_hbm, out_hbm, idx_v, row_v, rows_v, pos_v,
        sem0, sem1, osem0, osem1):
    wid = lax.axis_index("s") * _NC + lax.axis_index("c")
    base = wid * _B_PER_W
    sems = (sem0, sem1)
    osems = (osem0, osem1)

    pltpu.sync_copy(pos_hbm, pos_v)

    def _stage(c, wait_out):
        b = c % 2
        cb = base + c * _CHUNK
        pltpu.sync_copy(x_hbm.at[pl.ds(cb, _CHUNK)], idx_v.at[b])

        def _to_rows(v, carry):
            t = idx_v[b, pl.ds(v * 16, 16)]
            r = (
                jnp.bitwise_and(t, jnp.int32(~(_TB - 1)))
                | lax.shift_left(jnp.bitwise_and(t, _TB // 4 - 1), 2)
                | jnp.bitwise_and(
                    lax.shift_right_logical(t, (_TB // 4).bit_length() - 1), 3)
            )
            row_v[b, pl.ds(v * 16, 16)] = r
            return carry

        lax.fori_loop(0, _CHUNK // 16, _to_rows, 0)

        if wait_out is not None:
            wait_out.wait()
        copies = []
        for j in range(_NG):
            copies.append(
                pltpu.async_copy(
                    tbl_hbm.at[row_v.at[b, pl.ds(j * _G, _G)]],
                    rows_v.at[b, pl.ds(j * _G, _G)],
                    sems[b],
                )
            )
        return copies

    pending = _stage(0, None)
    out_cp = [None, None]
    for c in range(_NCHUNK):
        b = c % 2
        cb = base + c * _CHUNK
        l = cb >> 12
        q = lax.shift_right_logical(cb, 10) & 3

        nxt = None
        if c + 1 < _NCHUNK:
            nxt = _stage(c + 1, out_cp[(c + 1) % 2])

        for cp in pending:
            cp.wait()
        pending = nxt

        pv0 = pos_v[pl.ds(l * EMBED, 16)]
        pv1 = pos_v[pl.ds(l * EMBED + 16, 16)]

        def _add_pos(g, carry):
            for s in range(16):
                r = g * 16 + s
                plsc.addupdate(rows_v.at[b, r, pl.ds(0, 16)], pv0)
                plsc.addupdate(rows_v.at[b, r, pl.ds(16, 16)], pv1)
            return carry

        lax.fori_loop(0, _CHUNK // 16, _add_pos, 0)

        out_cp[b] = pltpu.async_copy(
            rows_v.at[b],
            out_hbm.at[pl.ds(l * 1024, 1024), pl.ds(q * EMBED, EMBED)],
            osems[b],
        )

    for cp in out_cp:
        if cp is not None:
            cp.wait()


_LPB = 4


def _t3_body(gin_ref, out_ref):
    eye = jnp.eye(EMBED, dtype=jnp.float32)
    for i in range(_LPB):
        g = gin_ref[pl.ds(i * 1024, 1024), :]
        toks = jnp.concatenate(
            [g[:, q * EMBED:(q + 1) * EMBED] for q in range(4)], axis=0)
        t = jax.lax.dot_general(eye, toks, (((1,), (1,)), ((), ())),
                                preferred_element_type=jnp.float32)
        r = t.reshape(EMBED // 8, 8, BATCH // 128, 128)
        out_ref[i] = r.transpose(0, 2, 1, 3)


_t3 = pl.pallas_call(
    _t3_body,
    grid=(MAXLEN // _LPB,),
    in_specs=[
        pl.BlockSpec((_LPB * BATCH * EMBED // 128, 128), lambda l: (l, 0)),
    ],
    out_specs=pl.BlockSpec(
        (_LPB, EMBED // 8, BATCH // 128, 8, 128), lambda l: (l, 0, 0, 0, 0)),
    out_shape=jax.ShapeDtypeStruct(
        (MAXLEN, EMBED // 8, BATCH // 128, 8, 128), jnp.float32),
)


def kernel(x, token_table, pos_table):
    tbl = _t1(token_table.T).reshape(_TROWS * 4, EMBED)
    xf = x.T.astype(jnp.int32).reshape(_ROWS)
    posf = pos_table.reshape(MAXLEN * EMBED)
    g = _k2(xf, tbl, posf)
    out5 = _t3(g)
    return out5.transpose(2, 4, 0, 1, 3).reshape(BATCH, MAXLEN, EMBED)


# speedup vs baseline: 2.4869x; 1.0053x over previous
import functools

import jax
import jax.numpy as jnp
from jax import lax
from jax.experimental import pallas as pl
from jax.experimental.pallas import tpu as pltpu
from jax.experimental.pallas import tpu_sc as plsc

VOCAB = 1000000
MAXLEN = 200
EMBED = 32
BATCH = 4096

_ROWS = BATCH * MAXLEN
_G = 128
_TB = 32768
_NTB = pl.cdiv(VOCAB, _TB)
_TROWS = _NTB * (_TB // 4)

_info = plsc.get_sparse_core_info()
_NC, _NS = _info.num_cores, _info.num_subcores
_NW = _NC * _NS
_B_PER_W = _ROWS // _NW
_CHUNK = 1024
_NCHUNK = _B_PER_W // _CHUNK
_NG = _CHUNK // _G


def _t1_body(tin_ref, tout_ref):
    eye = jnp.eye(EMBED, dtype=jnp.float32)
    qn = _TB // 4
    parts = []
    for q in range(4):
        a = tin_ref[:, pl.ds(q * qn, qn)]
        parts.append(
            jax.lax.dot_general(a, eye, (((0,), (0,)), ((), ())),
                                preferred_element_type=jnp.float32))
    tout_ref[...] = jnp.concatenate(parts, axis=1)


_t1 = pl.pallas_call(
    _t1_body,
    grid=(_NTB,),
    in_specs=[pl.BlockSpec((EMBED, _TB), lambda j: (0, j))],
    out_specs=pl.BlockSpec((_TB // 4, 128), lambda j: (j, 0)),
    out_shape=jax.ShapeDtypeStruct((_TROWS, 128), jnp.float32),
)


@functools.partial(
    pl.kernel,
    out_type=jax.ShapeDtypeStruct((_ROWS * EMBED // 128, 128), jnp.float32),
    mesh=plsc.VectorSubcoreMesh(core_axis_name="c", subcore_axis_name="s"),
    compiler_params=pltpu.CompilerParams(use_tc_tiling_on_sc=False),
    scratch_types=[
        pltpu.VMEM((2, _CHUNK), jnp.int32),
        pltpu.VMEM((2, _CHUNK), jnp.int32),
        pltpu.VMEM((2, _CHUNK, EMBED), jnp.float32),
        pltpu.VMEM((MAXLEN * EMBED,), jnp.float32),
        pltpu.SemaphoreType.DMA,
        pltpu.SemaphoreType.DMA,
        pltpu.SemaphoreType.DMA,
        pltpu.SemaphoreType.DMA,
    ],
)
def _k2(x_hbm, tbl_hbm, pos_hbm, out_hbm, idx_v, row_v, rows_v, pos_v,
        sem0, sem1, osem0, osem1):
    wid = lax.axis_index("s") * _NC + lax.axis_index("c")
    base = wid * _B_PER_W
    sems = (sem0, sem1)
    osems = (osem0, osem1)

    pltpu.sync_copy(pos_hbm, pos_v)

    def _stage(c, wait_out):
        b = c % 2
        cb = base + c * _CHUNK
        pltpu.sync_copy(x_hbm.at[pl.ds(cb, _CHUNK)], idx_v.at[b])

        def _to_rows(v, carry):
            t = idx_v[b, pl.ds(v * 16, 16)]
            r = (
                jnp.bitwise_and(t, jnp.int32(~(_TB - 1)))
                | lax.shift_left(jnp.bitwise_and(t, _TB // 4 - 1), 2)
                | jnp.bitwise_and(
                    lax.shift_right_logical(t, (_TB // 4).bit_length() - 1), 3)
            )
            row_v[b, pl.ds(v * 16, 16)] = r
            return carry

        lax.fori_loop(0, _CHUNK // 16, _to_rows, 0)

        if wait_out is not None:
            wait_out.wait()
        copies = []
        for j in range(_NG):
            copies.append(
                pltpu.async_copy(
                    tbl_hbm.at[row_v.at[b, pl.ds(j * _G, _G)]],
                    rows_v.at[b, pl.ds(j * _G, _G)],
                    sems[b],
                )
            )
        return copies

    pending = _stage(0, None)
    out_cp = [None, None]
    for c in range(_NCHUNK):
        b = c % 2
        cb = base + c * _CHUNK
        l = cb >> 12
        q = lax.shift_right_logical(cb, 10) & 3

        nxt = None
        if c + 1 < _NCHUNK:
            nxt = _stage(c + 1, out_cp[(c + 1) % 2])

        for cp in pending:
            cp.wait()
        pending = nxt

        pv0 = pos_v[pl.ds(l * EMBED, 16)]
        pv1 = pos_v[pl.ds(l * EMBED + 16, 16)]

        def _add_pos(g, carry):
            for s in range(16):
                r = g * 16 + s
                plsc.addupdate(rows_v.at[b, r, pl.ds(0, 16)], pv0)
                plsc.addupdate(rows_v.at[b, r, pl.ds(16, 16)], pv1)
            return carry

        lax.fori_loop(0, _CHUNK // 16, _add_pos, 0)

        out_cp[b] = pltpu.async_copy(
            rows_v.at[b],
            out_hbm.at[pl.ds(l * 1024, 1024), pl.ds(q * EMBED, EMBED)],
            osems[b],
        )

    for cp in out_cp:
        if cp is not None:
            cp.wait()


_LPB = 4


def _t3_body(gin_ref, out_ref):
    eye = jnp.eye(EMBED, dtype=jnp.float32)
    for i in range(_LPB):
        g = gin_ref[pl.ds(i * 1024, 1024), :]
        toks = jnp.concatenate(
            [g[:, q * EMBED:(q + 1) * EMBED] for q in range(4)], axis=0)
        t = jax.lax.dot_general(eye, toks, (((1,), (1,)), ((), ())),
                                preferred_element_type=jnp.float32)
        r = t.reshape(EMBED // 8, 8, BATCH // 128, 128)
        out_ref[i] = r.transpose(0, 2, 1, 3)


_t3 = pl.pallas_call(
    _t3_body,
    grid=(MAXLEN // _LPB,),
    in_specs=[
        pl.BlockSpec((_LPB * BATCH * EMBED // 128, 128), lambda l: (l, 0)),
    ],
    out_specs=pl.BlockSpec(
        (_LPB, EMBED // 8, BATCH // 128, 8, 128), lambda l: (l, 0, 0, 0, 0)),
    out_shape=jax.ShapeDtypeStruct(
        (MAXLEN, EMBED // 8, BATCH // 128, 8, 128), jnp.float32),
)


def kernel(x, token_table, pos_table):
    tbl = _t1(token_table.T).reshape(_TROWS * 4, EMBED)
    xf = x.T.astype(jnp.int32).reshape(_ROWS)
    posf = pos_table.reshape(MAXLEN * EMBED)
    g = _k2(xf, tbl, posf)
    out5 = _t3(g)
    return out5.transpose(2, 4, 0, 1, 3).reshape(BATCH, MAXLEN, EMBED)
